# baseline (speedup 1.0000x reference)
import functools

import jax
import jax.numpy as jnp
from jax import lax
from jax.experimental import pallas as pl
from jax.experimental.pallas import tpu as pltpu
from jax.experimental.pallas import tpu_sc as plsc

_EMB = 64
_EMBP = 128
_NW = 32
_CHUNK = 128
_NBUF = 5
_L = 16


@functools.partial(jax.jit, static_argnames=("seq", "batch"))
def _sc_gather(idx, table2, seq, batch):
    n = seq * batch
    nchunk_w = n // (_NW * _CHUNK)
    ng = nchunk_w // _NBUF
    assert ng * _NBUF == nchunk_w and ng >= 3
    mesh = plsc.VectorSubcoreMesh(core_axis_name="c", subcore_axis_name="s")

    @functools.partial(
        pl.kernel,
        out_type=jax.ShapeDtypeStruct((seq, 8, batch // _CHUNK, 8, _CHUNK), jnp.float32),
        mesh=mesh,
        scratch_types=[
            pltpu.VMEM((nchunk_w, _CHUNK), jnp.int32),
            [pltpu.VMEM((_CHUNK, _EMB), jnp.float32)] * _NBUF,
            [pltpu.VMEM((8, 8, _CHUNK), jnp.float32)] * _NBUF,
            [pltpu.SemaphoreType.DMA] * _NBUF,
            [pltpu.SemaphoreType.DMA] * _NBUF,
        ],
        compiler_params=pltpu.CompilerParams(
            use_tc_tiling_on_sc=False, needs_layout_passes=False
        ),
    )
    def k(idx_hbm, table_hbm, out_hbm, idx_v, gbufs, tbufs, gs, ws):
        wid = lax.axis_index("s") * 2 + lax.axis_index("c")
        base = wid * nchunk_w
        pltpu.sync_copy(idx_hbm.at[pl.ds(base, nchunk_w)], idx_v)

        lanes = lax.iota(jnp.int32, _L)
        row_idx = [lanes + tg * _L for tg in range(_CHUNK // _L)]

        def issue_gather(j, b):
            pltpu.async_copy(table_hbm.at[idx_v.at[j]], gbufs[b], gs[b])

        def wait_gather(b):
            pltpu.make_async_copy(
                table_hbm.at[pl.ds(0, _CHUNK)], gbufs[b], gs[b]
            ).wait()

        def out_slice(j):
            t0 = (base + j) * _CHUNK
            s = t0 // batch
            bb = (t0 % batch) // _CHUNK
            return out_hbm.at[s, pl.ds(0, 8), bb]

        def issue_write(j, b):
            pltpu.async_copy(tbufs[b], out_slice(j), ws[b])

        def wait_write(b):
            pltpu.make_async_copy(tbufs[b], out_slice(0), ws[b]).wait()

        def transpose(b):
            gbuf, tbuf = gbufs[b], tbufs[b]

            @pl.loop(0, _L)
            def _(d):
                perm = (lanes + d) & (_L - 1)
                for cg in range(_EMB // _L):
                    e_vec = perm + cg * _L
                    eb_vec = e_vec // 8
                    ei_vec = e_vec % 8
                    for tg in range(_CHUNK // _L):
                        vals = plsc.load_gather(gbuf, [row_idx[tg], e_vec])
                        plsc.store_scatter(tbuf, [eb_vec, ei_vec, row_idx[tg]], vals)

        for b in range(_NBUF):
            issue_gather(b, b)

        for b in range(_NBUF):
            wait_gather(b)
            transpose(b)
            issue_write(b, b)
            issue_gather(_NBUF + b, b)

        @pl.loop(1, ng - 1)
        def _(kk):
            j0 = kk * _NBUF
            for b in range(_NBUF):
                wait_gather(b)
                wait_write(b)
                transpose(b)
                issue_write(j0 + b, b)
                issue_gather(j0 + _NBUF + b, b)

        j0 = nchunk_w - _NBUF
        for b in range(_NBUF):
            wait_gather(b)
            wait_write(b)
            transpose(b)
            issue_write(j0 + b, b)
        for b in range(_NBUF):
            wait_write(b)

    return k(idx, table2)


def kernel(input, table):
    seq, batch = input.shape
    n = seq * batch
    idx = (input.astype(jnp.int32) * 2).reshape(n // _CHUNK, _CHUNK)
    table_p = jnp.pad(table.astype(jnp.float32), ((0, 0), (0, _EMBP - _EMB)))
    table2 = table_p.reshape(2 * table.shape[0], _EMB)
    out = _sc_gather(idx, table2, seq, batch)
    r = out.transpose(0, 2, 4, 1, 3)
    return r.reshape(seq, batch, _EMB)


# speedup vs baseline: 1.2507x; 1.2507x over previous
import functools

import jax
import jax.numpy as jnp
from jax import lax
from jax.experimental import pallas as pl
from jax.experimental.pallas import tpu as pltpu
from jax.experimental.pallas import tpu_sc as plsc

_EMB = 64
_EMBP = 128
_NW = 32
_CHUNK = 128
_NBUF = 5


@functools.partial(jax.jit, static_argnames=("n",))
def _sc_gather(idx, table2, n):
    nchunk_w = n // (_NW * _CHUNK)
    assert nchunk_w % _NBUF == 0 and nchunk_w >= 2 * _NBUF
    mesh = plsc.VectorSubcoreMesh(core_axis_name="c", subcore_axis_name="s")

    @functools.partial(
        pl.kernel,
        out_type=jax.ShapeDtypeStruct((n, _EMBP), jnp.float32),
        mesh=mesh,
        scratch_types=[
            pltpu.VMEM((nchunk_w, _CHUNK), jnp.int32),
            [pltpu.VMEM((_CHUNK, _EMB), jnp.float32)] * _NBUF,
            [pltpu.SemaphoreType.DMA] * _NBUF,
            [pltpu.SemaphoreType.DMA] * _NBUF,
        ],
        compiler_params=pltpu.CompilerParams(use_tc_tiling_on_sc=False),
    )
    def k(idx_hbm, table_hbm, out_hbm, idx_v, bufs, gs, ws):
        wid = lax.axis_index("s") * 2 + lax.axis_index("c")
        base = wid * (nchunk_w * _CHUNK)
        pltpu.sync_copy(idx_hbm.at[pl.ds(wid * nchunk_w, nchunk_w)], idx_v)

        def out_slice(j):
            return out_hbm.at[pl.ds(base + j * _CHUNK, _CHUNK), pl.ds(0, _EMB)]

        def issue_gather(j, b):
            pltpu.async_copy(table_hbm.at[idx_v.at[j]], bufs[b], gs[b])

        def wait_gather(b):
            pltpu.make_async_copy(table_hbm.at[pl.ds(0, _CHUNK)], bufs[b], gs[b]).wait()

        def issue_write(j, b):
            pltpu.async_copy(bufs[b], out_slice(j), ws[b])

        def wait_write(b):
            pltpu.make_async_copy(bufs[b], out_slice(0), ws[b]).wait()

        for b in range(_NBUF):
            issue_gather(b, b)

        @pl.loop(0, nchunk_w // _NBUF - 1)
        def _(kk):
            j0 = kk * _NBUF
            for b in range(_NBUF):
                wait_gather(b)
                issue_write(j0 + b, b)
                wait_write(b)
                issue_gather(j0 + _NBUF + b, b)

        j0 = nchunk_w - _NBUF
        for b in range(_NBUF):
            wait_gather(b)
            issue_write(j0 + b, b)
        for b in range(_NBUF):
            wait_write(b)

    return k(idx, table2)


def kernel(input, table):
    seq, batch = input.shape
    n = seq * batch
    idx = (input.astype(jnp.int32) * 2).reshape(n // _CHUNK, _CHUNK)
    table_p = jax.lax.dynamic_update_slice(
        jnp.zeros((table.shape[0], _EMBP), jnp.float32),
        table.astype(jnp.float32), (0, 0))
    table2 = table_p.reshape(2 * table.shape[0], _EMB)
    out = _sc_gather(idx, table2, n)
    return out[:, :_EMB].reshape(seq, batch, _EMB)


# speedup vs baseline: 1.2528x; 1.0017x over previous
import functools

import jax
import jax.numpy as jnp
from jax import lax
from jax.experimental import pallas as pl
from jax.experimental.pallas import tpu as pltpu
from jax.experimental.pallas import tpu_sc as plsc

_EMB = 64
_EMBP = 128
_NW = 32
_CHUNK = 128
_NBUF = 5


@functools.partial(jax.jit, static_argnames=("n",))
def _sc_gather(idx, table2, n):
    nchunk_w = n // (_NW * _CHUNK)
    assert nchunk_w % _NBUF == 0 and nchunk_w >= 2 * _NBUF
    mesh = plsc.VectorSubcoreMesh(core_axis_name="c", subcore_axis_name="s")

    @functools.partial(
        pl.kernel,
        out_type=jax.ShapeDtypeStruct((n, _EMBP), jnp.float32),
        mesh=mesh,
        scratch_types=[
            pltpu.VMEM((nchunk_w, _CHUNK), jnp.int32),
            [pltpu.VMEM((_CHUNK, _EMB), jnp.float32)] * _NBUF,
            [pltpu.SemaphoreType.DMA] * _NBUF,
            [pltpu.SemaphoreType.DMA] * _NBUF,
        ],
        compiler_params=pltpu.CompilerParams(use_tc_tiling_on_sc=False),
    )
    def k(idx_hbm, table_hbm, out_hbm, idx_v, bufs, gs, ws):
        wid = lax.axis_index("s") * 2 + lax.axis_index("c")
        base = wid * (nchunk_w * _CHUNK)
        pltpu.sync_copy(idx_hbm.at[pl.ds(wid * nchunk_w, nchunk_w)], idx_v)

        def out_slice(j):
            return out_hbm.at[pl.ds(base + j * _CHUNK, _CHUNK), pl.ds(0, _EMB)]

        def issue_gather(j, b):
            pltpu.async_copy(table_hbm.at[idx_v.at[j]], bufs[b], gs[b])

        def wait_gather(b):
            pltpu.make_async_copy(table_hbm.at[pl.ds(0, _CHUNK)], bufs[b], gs[b]).wait()

        def issue_write(j, b):
            pltpu.async_copy(bufs[b], out_slice(j), ws[b])

        def wait_write(b):
            pltpu.make_async_copy(bufs[b], out_slice(0), ws[b]).wait()

        for b in range(_NBUF):
            issue_gather(b, b)

        @pl.loop(0, nchunk_w // _NBUF - 1)
        def _(kk):
            j0 = kk * _NBUF
            for b in range(_NBUF):
                wait_gather(b)
                issue_write(j0 + b, b)
                wait_write(b)
                issue_gather(j0 + _NBUF + b, b)

        j0 = nchunk_w - _NBUF
        for b in range(_NBUF):
            wait_gather(b)
            issue_write(j0 + b, b)
        for b in range(_NBUF):
            wait_write(b)

    return k(idx, table2)


def kernel(input, table):
    seq, batch = input.shape
    n = seq * batch
    idx = (input.astype(jnp.int32) * 2).reshape(n // _CHUNK, _CHUNK)
    table_p = jnp.pad(table.astype(jnp.float32), ((0, 0), (0, _EMBP - _EMB)))
    table2 = table_p.reshape(2 * table.shape[0], _EMB)
    out = _sc_gather(idx, table2, n)
    return out[:, :_EMB].reshape(seq, batch, _EMB)
